# baseline (speedup 1.0000x reference)
import jax
import jax.numpy as jnp
from jax import lax
from jax.experimental import pallas as pl
from jax.experimental.pallas import tpu as pltpu
from jax.experimental.pallas import tpu_sc as plsc

_FIELD_DIM = 38461
_NF = 26
_CONT = 13
_BATCH = 16384
_NUM_CORES = 2
_NW = 32
_BPW = _BATCH // _NW
_L = 16
_CHUNK = 128
_BLK = 128
_NBLK = _BPW // _BLK
_BLK_ELEMS = _BLK * _NF
_CHUNKS_PER_BLK = _BLK_ELEMS // _CHUNK


def _sc_body(x_h, xc_h, table_h, bias_h, w_h, out_h,
             xf_v, xcf_v, idx_v, g_v, w_v, b_v, out_v,
             sem_xc, sem0, sem1, sem2, sem3):
    c = lax.axis_index("c")
    s = lax.axis_index("s")
    wid = s * _NUM_CORES + c
    base = wid * _BPW
    sems = [sem0, sem1, sem2, sem3]

    xc_copy = pltpu.make_async_copy(
        xc_h.at[pl.ds(base * _CONT, _BPW * _CONT)], xcf_v, sem_xc)
    xc_copy.start()
    pltpu.sync_copy(w_h, w_v.at[pl.ds(0, _CONT)])
    pltpu.sync_copy(bias_h, b_v.at[pl.ds(0, 1)])
    pltpu.sync_copy(x_h.at[pl.ds(base * _NF, _BPW * _NF)], xf_v)

    iota = lax.iota(jnp.int32, _L)

    for j in range(_NBLK):
        jb = j * _BLK_ELEMS

        @pl.loop(0, _BLK_ELEMS // _L)
        def _idx_loop(k, jb=jb):
            o = jb + k * _L
            p = iota + o
            f = lax.rem(p, _NF)
            idx_v[pl.ds(o, _L)] = xf_v[pl.ds(o, _L)] + f * _FIELD_DIM

        @pl.loop(0, _CHUNKS_PER_BLK)
        def _fire(r, jb=jb, sem=sems[j]):
            o = jb + r * _CHUNK
            pltpu.make_async_copy(
                table_h.at[idx_v.at[pl.ds(o, _CHUNK)]],
                g_v.at[pl.ds(o, _CHUNK)], sem,
            ).start()

    w_vec = w_v[...]
    w_s = [w_vec[i] for i in range(_CONT)]
    b_vec = b_v[...]
    bias_s = b_vec[0]
    xc_copy.wait()
    iota26 = iota * _NF
    iota13 = iota * _CONT

    for j in range(_NBLK):
        jb = j * _BLK_ELEMS

        @pl.loop(0, _CHUNKS_PER_BLK)
        def _drain(r, jb=jb, sem=sems[j]):
            o = jb + r * _CHUNK
            pltpu.make_async_copy(
                table_h.at[idx_v.at[pl.ds(o, _CHUNK)]],
                g_v.at[pl.ds(o, _CHUNK)], sem,
            ).wait()

        @pl.loop(0, _BLK // _L)
        def _acc_loop(k, j=j):
            b0 = j * _BLK + k * _L
            acc = jnp.full((_L,), bias_s, jnp.float32)
            gbase = iota26 + b0 * _NF
            for f in range(_NF):
                acc = acc + plsc.load_gather(g_v, [gbase + f])
            cbase = iota13 + b0 * _CONT
            for cc in range(_CONT):
                acc = acc + plsc.load_gather(xcf_v, [cbase + cc]) * w_s[cc]
            out_v[pl.ds(b0, _L)] = acc

    pltpu.sync_copy(out_v, out_h.at[pl.ds(base, _BPW)])


def _make_kernel():
    mesh = plsc.VectorSubcoreMesh(core_axis_name="c", subcore_axis_name="s")
    return pl.kernel(
        _sc_body,
        out_type=jax.ShapeDtypeStruct((_BATCH,), jnp.float32),
        mesh=mesh,
        scratch_types=[
            pltpu.VMEM((_BPW * _NF,), jnp.int32),
            pltpu.VMEM((_BPW * _CONT,), jnp.float32),
            pltpu.VMEM((_BPW * _NF,), jnp.int32),
            pltpu.VMEM((_BPW * _NF,), jnp.float32),
            pltpu.VMEM((_L,), jnp.float32),
            pltpu.VMEM((_L,), jnp.float32),
            pltpu.VMEM((_BPW,), jnp.float32),
            pltpu.SemaphoreType.DMA,
            pltpu.SemaphoreType.DMA,
            pltpu.SemaphoreType.DMA,
            pltpu.SemaphoreType.DMA,
            pltpu.SemaphoreType.DMA,
        ],
        compiler_params=pltpu.CompilerParams(needs_layout_passes=False),
    )


_fm_linear_sc = _make_kernel()


@jax.jit
def kernel(x, x_cont, table, bias, w):
    out = _fm_linear_sc(
        x.reshape(-1), x_cont.reshape(-1), table.reshape(-1), bias, w
    )
    return out.reshape(-1, 1)


# speedup vs baseline: 2.4119x; 2.4119x over previous
import jax
import jax.numpy as jnp
from jax import lax
from jax.experimental import pallas as pl
from jax.experimental.pallas import tpu as pltpu
from jax.experimental.pallas import tpu_sc as plsc

_FIELD_DIM = 38461
_NF = 26
_CONT = 13
_BATCH = 16384
_VOCAB = _FIELD_DIM * _NF
_NUM_CORES = 2
_NW = 32
_BPW = _BATCH // _NW
_L = 16
_CHUNK = 128
_BLK = 128
_NBLK = _BPW // _BLK


_VPAD = ((_VOCAB + 1023) // 1024) * 1024


def _sc_body(xt_h, xct_h, table_h, bias_h, w_h, out_h,
             xt_v, xct_v, idx_v, g_v, w_v, b_v, out_v,
             sem_in, sem_xc, sem0, sem1, sem2, sem3):
    c = lax.axis_index("c")
    s = lax.axis_index("s")
    wid = s * _NUM_CORES + c
    base = wid * _BPW
    sems = [sem0, sem1, sem2, sem3]

    in_copy = pltpu.make_async_copy(
        xt_h.at[:, pl.ds(base, _BPW)], xt_v, sem_in)
    in_copy.start()
    xc_copy = pltpu.make_async_copy(
        xct_h.at[:, pl.ds(base, _BPW)], xct_v, sem_xc)
    xc_copy.start()
    pltpu.sync_copy(w_h, w_v.at[pl.ds(0, _CONT)])
    pltpu.sync_copy(bias_h, b_v.at[pl.ds(0, 1)])
    in_copy.wait()

    for j in range(_NBLK):
        jo = j * _BLK

        @pl.loop(0, _NF)
        def _idx_fire(f, jo=jo, sem=sems[j]):
            fo = f * _FIELD_DIM
            o = f * _BPW + jo
            for k in range(_BLK // _L):
                xv = xt_v[f, pl.ds(jo + k * _L, _L)]
                idx_v[pl.ds(o + k * _L, _L)] = xv + fo
            pltpu.make_async_copy(
                table_h.at[idx_v.at[pl.ds(o, _CHUNK)]],
                g_v.at[pl.ds(o, _CHUNK)], sem,
            ).start()

    w_vec = w_v[...]
    w_s = [w_vec[i] for i in range(_CONT)]
    bias_s = b_v[...][0]
    xc_copy.wait()

    for j in range(_NBLK):
        jo = j * _BLK

        @pl.loop(0, _NF)
        def _drain(f, jo=jo, sem=sems[j]):
            o = f * _BPW + jo
            pltpu.make_async_copy(
                table_h.at[idx_v.at[pl.ds(o, _CHUNK)]],
                g_v.at[pl.ds(o, _CHUNK)], sem,
            ).wait()

        @pl.loop(0, _BLK // _L)
        def _acc_loop(k, jo=jo):
            o = jo + k * _L
            acc = jnp.full((_L,), bias_s, jnp.float32)
            for f in range(_NF):
                acc = acc + g_v[pl.ds(o + f * _BPW, _L)]
            for cc in range(_CONT):
                acc = acc + xct_v[cc, pl.ds(o, _L)] * w_s[cc]
            out_v[pl.ds(o, _L)] = acc

    pltpu.sync_copy(out_v, out_h.at[pl.ds(base, _BPW)])


def _make_kernel():
    mesh = plsc.VectorSubcoreMesh(core_axis_name="c", subcore_axis_name="s")
    return pl.kernel(
        _sc_body,
        out_type=jax.ShapeDtypeStruct((_BATCH,), jnp.float32),
        mesh=mesh,
        scratch_types=[
            pltpu.VMEM((_NF, _BPW), jnp.int32),
            pltpu.VMEM((_CONT, _BPW), jnp.float32),
            pltpu.VMEM((_NF * _BPW,), jnp.int32),
            pltpu.VMEM((_NF * _BPW,), jnp.float32),
            pltpu.VMEM((_L,), jnp.float32),
            pltpu.VMEM((_L,), jnp.float32),
            pltpu.VMEM((_BPW,), jnp.float32),
            pltpu.SemaphoreType.DMA,
            pltpu.SemaphoreType.DMA,
            pltpu.SemaphoreType.DMA,
            pltpu.SemaphoreType.DMA,
            pltpu.SemaphoreType.DMA,
            pltpu.SemaphoreType.DMA,
        ],
        compiler_params=pltpu.CompilerParams(needs_layout_passes=False),
    )


_fm_linear_sc = _make_kernel()


@jax.jit
def kernel(x, x_cont, table, bias, w):
    tab = jnp.pad(table, ((0, _VPAD - _VOCAB), (0, 0))).reshape(-1)
    out = _fm_linear_sc(x.T, x_cont.T, tab, bias, w)
    return out.reshape(-1, 1)


# speedup vs baseline: 2.4512x; 1.0163x over previous
import jax
import jax.numpy as jnp
from jax import lax
from jax.experimental import pallas as pl
from jax.experimental.pallas import tpu as pltpu
from jax.experimental.pallas import tpu_sc as plsc

_FIELD_DIM = 38461
_NF = 26
_CONT = 13
_BATCH = 16384
_VOCAB = _FIELD_DIM * _NF
_NUM_CORES = 2
_NW = 32
_BPW = _BATCH // _NW
_L = 16
_CHUNK = 128
_BLK = 128
_NBLK = _BPW // _BLK


_VMAIN = (_VOCAB // 1024) * 1024
_VTAIL = _VOCAB - _VMAIN
_TAILPAD = 1024


def _sc_body(xt_h, xct_h, table_h, tail_h, bias_h, w_h, out_h,
             xt_v, xct_v, idx_v, g_v, tail_v, w_v, b_v, out_v,
             sem_in, sem_xc, sem0, sem1, sem2, sem3):
    c = lax.axis_index("c")
    s = lax.axis_index("s")
    wid = s * _NUM_CORES + c
    base = wid * _BPW
    sems = [sem0, sem1, sem2, sem3]

    in_copy = pltpu.make_async_copy(
        xt_h.at[:, pl.ds(base, _BPW)], xt_v, sem_in)
    in_copy.start()
    xc_copy = pltpu.make_async_copy(
        xct_h.at[:, pl.ds(base, _BPW)], xct_v, sem_xc)
    xc_copy.start()
    pltpu.sync_copy(w_h, w_v.at[pl.ds(0, _CONT)])
    pltpu.sync_copy(bias_h, b_v.at[pl.ds(0, 1)])
    pltpu.sync_copy(tail_h, tail_v)
    in_copy.wait()

    for j in range(_NBLK):
        jo = j * _BLK

        @pl.loop(0, _NF)
        def _idx_fire(f, jo=jo, sem=sems[j]):
            fo = f * _FIELD_DIM
            o = f * _BPW + jo
            for k in range(_BLK // _L):
                xv = xt_v[f, pl.ds(jo + k * _L, _L)]
                idx_v[pl.ds(o + k * _L, _L)] = jnp.minimum(
                    xv + fo, _VMAIN - 1)
            pltpu.make_async_copy(
                table_h.at[idx_v.at[pl.ds(o, _CHUNK)]],
                g_v.at[pl.ds(o, _CHUNK)], sem,
            ).start()

    w_vec = w_v[...]
    w_s = [w_vec[i] for i in range(_CONT)]
    bias_s = b_v[...][0]
    xc_copy.wait()

    for j in range(_NBLK):
        jo = j * _BLK

        @pl.loop(0, _NF)
        def _drain(f, jo=jo, sem=sems[j]):
            o = f * _BPW + jo
            pltpu.make_async_copy(
                table_h.at[idx_v.at[pl.ds(o, _CHUNK)]],
                g_v.at[pl.ds(o, _CHUNK)], sem,
            ).wait()

        @pl.loop(0, _BLK // _L)
        def _acc_loop(k, jo=jo):
            o = jo + k * _L
            acc = jnp.full((_L,), bias_s, jnp.float32)
            for f in range(_NF - 1):
                acc = acc + g_v[pl.ds(o + f * _BPW, _L)]
            raw25 = xt_v[_NF - 1, pl.ds(o, _L)] + (_NF - 1) * _FIELD_DIM
            toff = jnp.clip(raw25 - _VMAIN, 0, _TAILPAD - 1)
            tval = plsc.load_gather(tail_v, [toff])
            gval = g_v[pl.ds(o + (_NF - 1) * _BPW, _L)]
            acc = acc + jnp.where(raw25 >= _VMAIN, tval, gval)
            for cc in range(_CONT):
                acc = acc + xct_v[cc, pl.ds(o, _L)] * w_s[cc]
            out_v[pl.ds(o, _L)] = acc

    pltpu.sync_copy(out_v, out_h.at[pl.ds(base, _BPW)])


def _make_kernel():
    mesh = plsc.VectorSubcoreMesh(core_axis_name="c", subcore_axis_name="s")
    return pl.kernel(
        _sc_body,
        out_type=jax.ShapeDtypeStruct((_BATCH,), jnp.float32),
        mesh=mesh,
        scratch_types=[
            pltpu.VMEM((_NF, _BPW), jnp.int32),
            pltpu.VMEM((_CONT, _BPW), jnp.float32),
            pltpu.VMEM((_NF * _BPW,), jnp.int32),
            pltpu.VMEM((_NF * _BPW,), jnp.float32),
            pltpu.VMEM((_TAILPAD,), jnp.float32),
            pltpu.VMEM((_L,), jnp.float32),
            pltpu.VMEM((_L,), jnp.float32),
            pltpu.VMEM((_BPW,), jnp.float32),
            pltpu.SemaphoreType.DMA,
            pltpu.SemaphoreType.DMA,
            pltpu.SemaphoreType.DMA,
            pltpu.SemaphoreType.DMA,
            pltpu.SemaphoreType.DMA,
            pltpu.SemaphoreType.DMA,
        ],
        compiler_params=pltpu.CompilerParams(needs_layout_passes=False),
    )


_fm_linear_sc = _make_kernel()


@jax.jit
def kernel(x, x_cont, table, bias, w):
    tab_main = table[:_VMAIN, :].reshape(-1)
    tab_tail = jnp.pad(
        table[_VMAIN:, :], ((0, _TAILPAD - _VTAIL), (0, 0))
    ).reshape(-1)
    out = _fm_linear_sc(x.T, x_cont.T, tab_main, tab_tail, bias, w)
    return out.reshape(-1, 1)
